# baseline (speedup 1.0000x reference)
import jax
import jax.numpy as jnp
from jax import lax
from jax.experimental import pallas as pl
from jax.experimental.pallas import tpu as pltpu
from jax.experimental.pallas import tpu_sc as plsc

import numpy as np

_PATH_NUM = 16
_N = 8192
_D = 4096
_NUM_CORES = 2
_NUM_SUBCORES = 16
_NW = _NUM_CORES * _NUM_SUBCORES
_B_PER_W = _N // _NW
_CHUNK_SIZES = (8, 8)
_CHUNKS = []
_r = 0
while _r < _B_PER_W:
  _n = min(_CHUNK_SIZES[len(_CHUNKS) % 2], _B_PER_W - _r)
  _CHUNKS.append((_r, _n))
  _r += _n


_NBUF = 2


def _gather_body(inputs_hbm, order_hbm, out_hbm, idx_v, bufs, gsems, ssems):
  wid = lax.axis_index("s") * _NUM_CORES + lax.axis_index("c")
  base = wid * _B_PER_W
  pltpu.sync_copy(order_hbm.at[pl.ds(base, _B_PER_W)], idx_v)

  def start_gather(c, b):
    r0, n = _CHUNKS[c]
    idx_slice = idx_v.at[pl.ds(r0, n)]
    dst = bufs[b] if n == _CHUNK_SIZES[b] else bufs[b].at[pl.ds(0, n)]
    return pltpu.async_copy(inputs_hbm.at[idx_slice], dst, gsems[b])

  def start_scatter(c, b):
    r0, n = _CHUNKS[c]
    src = bufs[b] if n == _CHUNK_SIZES[b] else bufs[b].at[pl.ds(0, n)]
    dst = out_hbm.at[pl.ds(base + r0, n)]
    return pltpu.async_copy(src, dst, ssems[b])

  copies = [None] * _NBUF
  scats = [None] * _NBUF
  nck = len(_CHUNKS)
  for b in range(min(_NBUF, nck)):
    copies[b] = start_gather(b, b)
  for c in range(nck):
    b = c % _NBUF
    copies[b].wait()
    scats[b] = start_scatter(c, b)
    if c >= 1 and c + _NBUF - 1 < nck:
      pb = (c - 1) % _NBUF
      scats[pb].wait()
      copies[pb] = start_gather(c + _NBUF - 1, pb)
  for b in range(_NBUF):
    if scats[b] is not None:
      scats[b].wait()


@jax.jit
def _dispatch(inputs, order):
  mesh = plsc.VectorSubcoreMesh(core_axis_name="c", subcore_axis_name="s")
  f = pl.kernel(
      _gather_body,
      out_type=jax.ShapeDtypeStruct((_N, _D), jnp.float32),
      mesh=mesh,
      scratch_types=[
          pltpu.VMEM((_B_PER_W,), jnp.int32),
          [pltpu.VMEM((_CHUNK_SIZES[b], _D), jnp.float32) for b in range(_NBUF)],
          [pltpu.SemaphoreType.DMA for _ in range(_NBUF)],
          [pltpu.SemaphoreType.DMA for _ in range(_NBUF)],
      ],
  )
  return f(inputs, order)


def _threefry2x32_np(k1, k2, x0, x1):
  def rotl(x, d):
    return (x << np.uint32(d)) | (x >> np.uint32(32 - d))

  ks = [np.uint32(k1), np.uint32(k2),
        np.uint32(k1) ^ np.uint32(k2) ^ np.uint32(0x1BD11BDA)]
  x = [x0 + ks[0], x1 + ks[1]]
  r_even = (13, 15, 26, 6)
  r_odd = (17, 29, 16, 24)

  def rounds(x, rs):
    for r in rs:
      x[0] = x[0] + x[1]
      x[1] = x[0] ^ rotl(x[1], r)
    return x

  x = rounds(x, r_even); x[0] += ks[1]; x[1] += ks[2] + np.uint32(1)
  x = rounds(x, r_odd); x[0] += ks[2]; x[1] += ks[0] + np.uint32(2)
  x = rounds(x, r_even); x[0] += ks[0]; x[1] += ks[1] + np.uint32(3)
  x = rounds(x, r_odd); x[0] += ks[1]; x[1] += ks[2] + np.uint32(4)
  x = rounds(x, r_even); x[0] += ks[2]; x[1] += ks[0] + np.uint32(5)
  return x


def _routing_constants():
  with np.errstate(over="ignore"):
    c_lo = np.arange(_N * _PATH_NUM, dtype=np.uint32)
    b1, b2 = _threefry2x32_np(0, 42, np.zeros_like(c_lo), c_lo)
    bits = (b1 ^ b2).reshape(_N, _PATH_NUM)
  lo = np.float32(np.nextafter(np.float32(-1), np.float32(0)))
  hi = np.float32(1.0)
  mant = (bits >> np.uint32(9)) | np.uint32(0x3F800000)
  floats = mant.view(np.float32) - np.float32(1.0)
  u = np.maximum(lo, floats * (hi - lo) + lo)
  route = np.argmax(u, axis=1).astype(np.int32)
  order = np.argsort(route, kind="stable").astype(np.int32)
  route_sorted = route[order]
  counts = np.bincount(route, minlength=_PATH_NUM).astype(np.int32)
  return order, route_sorted, counts


_ORDER_NP, _ROUTE_SORTED_NP, _COUNTS_NP = _routing_constants()


def kernel(inputs):
  order = jnp.asarray(_ORDER_NP)
  route_sorted = jnp.asarray(_ROUTE_SORTED_NP)
  counts = jnp.asarray(_COUNTS_NP)
  dispatched = _dispatch(inputs, order)
  return dispatched, route_sorted, counts


# speedup vs baseline: 1.0161x; 1.0161x over previous
import jax
import jax.numpy as jnp
from jax import lax
from jax.experimental import pallas as pl
from jax.experimental.pallas import tpu as pltpu
from jax.experimental.pallas import tpu_sc as plsc

import numpy as np

_PATH_NUM = 16
_N = 8192
_D = 4096
_NUM_CORES = 2
_NUM_SUBCORES = 16
_NW = _NUM_CORES * _NUM_SUBCORES
_B_PER_W = _N // _NW
_CHUNK_SIZES = (8, 8)
_CHUNKS = []
_r = 0
while _r < _B_PER_W:
  _n = min(_CHUNK_SIZES[len(_CHUNKS) % 2], _B_PER_W - _r)
  _CHUNKS.append((_r, _n))
  _r += _n


_NBUF = 2


def _gather_body(inputs_hbm, order_hbm, out_hbm, idx_v, bufs, gsems, ssems):
  wid = lax.axis_index("s") * _NUM_CORES + lax.axis_index("c")
  base = wid * _B_PER_W
  pltpu.sync_copy(order_hbm.at[pl.ds(base, _B_PER_W)], idx_v)

  def start_gather(c, b):
    r0, n = _CHUNKS[c]
    idx_slice = idx_v.at[pl.ds(r0, n)]
    dst = bufs[b] if n == _CHUNK_SIZES[b] else bufs[b].at[pl.ds(0, n)]
    return pltpu.async_copy(inputs_hbm.at[idx_slice], dst, gsems[b])

  def start_scatter(c, b):
    r0, n = _CHUNKS[c]
    src = bufs[b] if n == _CHUNK_SIZES[b] else bufs[b].at[pl.ds(0, n)]
    dst = out_hbm.at[pl.ds(base + r0, n)]
    return pltpu.async_copy(src, dst, ssems[b])

  copies = [None] * _NBUF
  scats = [None] * _NBUF
  nck = len(_CHUNKS)
  copies[0] = start_gather(0, 0)
  for c in range(nck):
    b = c % _NBUF
    nb = (c + 1) % _NBUF
    if c + 1 < nck:
      if scats[nb] is not None:
        scats[nb].wait()
      copies[nb] = start_gather(c + 1, nb)
    copies[b].wait()
    scats[b] = start_scatter(c, b)
  for b in range(_NBUF):
    if scats[b] is not None:
      scats[b].wait()


@jax.jit
def _dispatch(inputs, order):
  mesh = plsc.VectorSubcoreMesh(core_axis_name="c", subcore_axis_name="s")
  f = pl.kernel(
      _gather_body,
      out_type=jax.ShapeDtypeStruct((_N, _D), jnp.float32),
      mesh=mesh,
      scratch_types=[
          pltpu.VMEM((_B_PER_W,), jnp.int32),
          [pltpu.VMEM((_CHUNK_SIZES[b], _D), jnp.float32) for b in range(_NBUF)],
          [pltpu.SemaphoreType.DMA for _ in range(_NBUF)],
          [pltpu.SemaphoreType.DMA for _ in range(_NBUF)],
      ],
  )
  return f(inputs, order)


def _threefry2x32_np(k1, k2, x0, x1):
  def rotl(x, d):
    return (x << np.uint32(d)) | (x >> np.uint32(32 - d))

  ks = [np.uint32(k1), np.uint32(k2),
        np.uint32(k1) ^ np.uint32(k2) ^ np.uint32(0x1BD11BDA)]
  x = [x0 + ks[0], x1 + ks[1]]
  r_even = (13, 15, 26, 6)
  r_odd = (17, 29, 16, 24)

  def rounds(x, rs):
    for r in rs:
      x[0] = x[0] + x[1]
      x[1] = x[0] ^ rotl(x[1], r)
    return x

  x = rounds(x, r_even); x[0] += ks[1]; x[1] += ks[2] + np.uint32(1)
  x = rounds(x, r_odd); x[0] += ks[2]; x[1] += ks[0] + np.uint32(2)
  x = rounds(x, r_even); x[0] += ks[0]; x[1] += ks[1] + np.uint32(3)
  x = rounds(x, r_odd); x[0] += ks[1]; x[1] += ks[2] + np.uint32(4)
  x = rounds(x, r_even); x[0] += ks[2]; x[1] += ks[0] + np.uint32(5)
  return x


def _routing_constants():
  with np.errstate(over="ignore"):
    c_lo = np.arange(_N * _PATH_NUM, dtype=np.uint32)
    b1, b2 = _threefry2x32_np(0, 42, np.zeros_like(c_lo), c_lo)
    bits = (b1 ^ b2).reshape(_N, _PATH_NUM)
  lo = np.float32(np.nextafter(np.float32(-1), np.float32(0)))
  hi = np.float32(1.0)
  mant = (bits >> np.uint32(9)) | np.uint32(0x3F800000)
  floats = mant.view(np.float32) - np.float32(1.0)
  u = np.maximum(lo, floats * (hi - lo) + lo)
  route = np.argmax(u, axis=1).astype(np.int32)
  order = np.argsort(route, kind="stable").astype(np.int32)
  route_sorted = route[order]
  counts = np.bincount(route, minlength=_PATH_NUM).astype(np.int32)
  return order, route_sorted, counts


_ORDER_NP, _ROUTE_SORTED_NP, _COUNTS_NP = _routing_constants()


def kernel(inputs):
  order = jnp.asarray(_ORDER_NP)
  route_sorted = jnp.asarray(_ROUTE_SORTED_NP)
  counts = jnp.asarray(_COUNTS_NP)
  dispatched = _dispatch(inputs, order)
  return dispatched, route_sorted, counts


# speedup vs baseline: 1.0189x; 1.0028x over previous
import jax
import jax.numpy as jnp
from jax import lax
from jax.experimental import pallas as pl
from jax.experimental.pallas import tpu as pltpu
from jax.experimental.pallas import tpu_sc as plsc

import numpy as np

_PATH_NUM = 16
_N = 8192
_D = 4096
_NUM_CORES = 2
_NUM_SUBCORES = 16
_NW = _NUM_CORES * _NUM_SUBCORES
_B_PER_W = _N // _NW
_CHUNK_SIZES = (8, 8, 8)
_CHUNKS = []
_r = 0
while _r < _B_PER_W:
  _n = min(_CHUNK_SIZES[len(_CHUNKS) % len(_CHUNK_SIZES)], _B_PER_W - _r)
  _CHUNKS.append((_r, _n))
  _r += _n


_NBUF = 3


def _gather_body(inputs_hbm, order_hbm, out_hbm, idx_v, bufs, gsems, ssems):
  wid = lax.axis_index("s") * _NUM_CORES + lax.axis_index("c")
  base = wid * _B_PER_W
  pltpu.sync_copy(order_hbm.at[pl.ds(base, _B_PER_W)], idx_v)

  def start_gather(c, b):
    r0, n = _CHUNKS[c]
    idx_slice = idx_v.at[pl.ds(r0, n)]
    dst = bufs[b] if n == _CHUNK_SIZES[b] else bufs[b].at[pl.ds(0, n)]
    return pltpu.async_copy(inputs_hbm.at[idx_slice], dst, gsems[b])

  def start_scatter(c, b):
    r0, n = _CHUNKS[c]
    src = bufs[b] if n == _CHUNK_SIZES[b] else bufs[b].at[pl.ds(0, n)]
    dst = out_hbm.at[pl.ds(base + r0, n)]
    return pltpu.async_copy(src, dst, ssems[b])

  copies = [None] * _NBUF
  scats = [None] * _NBUF
  nck = len(_CHUNKS)
  for b in range(min(_NBUF - 1, nck)):
    copies[b] = start_gather(b, b)
  for c in range(nck):
    b = c % _NBUF
    nxt = c + _NBUF - 1
    if nxt < nck:
      nb = nxt % _NBUF
      if scats[nb] is not None:
        scats[nb].wait()
      copies[nb] = start_gather(nxt, nb)
    copies[b].wait()
    scats[b] = start_scatter(c, b)
  for b in range(_NBUF):
    if scats[b] is not None:
      scats[b].wait()


@jax.jit
def _dispatch(inputs, order):
  mesh = plsc.VectorSubcoreMesh(core_axis_name="c", subcore_axis_name="s")
  f = pl.kernel(
      _gather_body,
      out_type=jax.ShapeDtypeStruct((_N, _D), jnp.float32),
      mesh=mesh,
      scratch_types=[
          pltpu.VMEM((_B_PER_W,), jnp.int32),
          [pltpu.VMEM((_CHUNK_SIZES[b], _D), jnp.float32) for b in range(_NBUF)],
          [pltpu.SemaphoreType.DMA for _ in range(_NBUF)],
          [pltpu.SemaphoreType.DMA for _ in range(_NBUF)],
      ],
  )
  return f(inputs, order)


def _threefry2x32_np(k1, k2, x0, x1):
  def rotl(x, d):
    return (x << np.uint32(d)) | (x >> np.uint32(32 - d))

  ks = [np.uint32(k1), np.uint32(k2),
        np.uint32(k1) ^ np.uint32(k2) ^ np.uint32(0x1BD11BDA)]
  x = [x0 + ks[0], x1 + ks[1]]
  r_even = (13, 15, 26, 6)
  r_odd = (17, 29, 16, 24)

  def rounds(x, rs):
    for r in rs:
      x[0] = x[0] + x[1]
      x[1] = x[0] ^ rotl(x[1], r)
    return x

  x = rounds(x, r_even); x[0] += ks[1]; x[1] += ks[2] + np.uint32(1)
  x = rounds(x, r_odd); x[0] += ks[2]; x[1] += ks[0] + np.uint32(2)
  x = rounds(x, r_even); x[0] += ks[0]; x[1] += ks[1] + np.uint32(3)
  x = rounds(x, r_odd); x[0] += ks[1]; x[1] += ks[2] + np.uint32(4)
  x = rounds(x, r_even); x[0] += ks[2]; x[1] += ks[0] + np.uint32(5)
  return x


def _routing_constants():
  with np.errstate(over="ignore"):
    c_lo = np.arange(_N * _PATH_NUM, dtype=np.uint32)
    b1, b2 = _threefry2x32_np(0, 42, np.zeros_like(c_lo), c_lo)
    bits = (b1 ^ b2).reshape(_N, _PATH_NUM)
  lo = np.float32(np.nextafter(np.float32(-1), np.float32(0)))
  hi = np.float32(1.0)
  mant = (bits >> np.uint32(9)) | np.uint32(0x3F800000)
  floats = mant.view(np.float32) - np.float32(1.0)
  u = np.maximum(lo, floats * (hi - lo) + lo)
  route = np.argmax(u, axis=1).astype(np.int32)
  order = np.argsort(route, kind="stable").astype(np.int32)
  route_sorted = route[order]
  counts = np.bincount(route, minlength=_PATH_NUM).astype(np.int32)
  return order, route_sorted, counts


_ORDER_NP, _ROUTE_SORTED_NP, _COUNTS_NP = _routing_constants()


def kernel(inputs):
  order = jnp.asarray(_ORDER_NP)
  route_sorted = jnp.asarray(_ROUTE_SORTED_NP)
  counts = jnp.asarray(_COUNTS_NP)
  dispatched = _dispatch(inputs, order)
  return dispatched, route_sorted, counts
